# baseline (speedup 1.0000x reference)
import functools

import jax
import jax.numpy as jnp
from jax import lax
from jax.experimental import pallas as pl
from jax.experimental.pallas import tpu as pltpu
from jax.experimental.pallas import tpu_sc as plsc

CH = 96
NW = 32
QR = 2
IR = 4
RI = 256


@functools.lru_cache(maxsize=None)
def _make_fnet(E, Epad, DE, H, D):
    BE = 2048
    grid = (Epad // BE,)

    def body(ea, w1, b1, w2, b2, w_out):
        h = jnp.tanh(jnp.dot(ea[...], w1[...], preferred_element_type=jnp.float32)
                     + b1[...])
        w = jnp.dot(h, w2[...], preferred_element_type=jnp.float32) + b2[...]
        i = pl.program_id(0)
        rows = i * BE + lax.broadcasted_iota(jnp.int32, (BE, 1), 0)
        w_out[...] = jnp.where(rows < E, w, 0.0)

    return pl.pallas_call(
        body,
        grid=grid,
        in_specs=[
            pl.BlockSpec((BE, DE), lambda i: (i, 0)),
            pl.BlockSpec((DE, H), lambda i: (0, 0)),
            pl.BlockSpec((1, H), lambda i: (0, 0)),
            pl.BlockSpec((H, D), lambda i: (0, 0)),
            pl.BlockSpec((1, D), lambda i: (0, 0)),
        ],
        out_specs=pl.BlockSpec((BE, D), lambda i: (i, 0)),
        out_shape=jax.ShapeDtypeStruct((Epad, D), jnp.float32),
    )


@functools.lru_cache(maxsize=None)
def _make_softmax(N, D, BN):
    def body(x, o):
        v = x[...]
        m = jnp.max(v, axis=-1, keepdims=True)
        e = jnp.exp(v - m)
        o[...] = e / jnp.sum(e, axis=-1, keepdims=True)

    return pl.pallas_call(
        body,
        grid=(N // BN,),
        in_specs=[pl.BlockSpec((BN, D), lambda i: (i, 0))],
        out_specs=pl.BlockSpec((BN, D), lambda i: (i, 0)),
        out_shape=jax.ShapeDtypeStruct((N, D), jnp.float32),
    )


@functools.lru_cache(maxsize=None)
def _make_update(N, D, BN, do_softmax):
    def body(x, p0, p1, d0, d1, o):
        deg = d0[...] + d1[...]
        degc = jnp.maximum(deg, 1.0)
        q = x[...] - (p0[...] + p1[...]) / degc
        if do_softmax:
            m = jnp.max(q, axis=-1, keepdims=True)
            e = jnp.exp(q - m)
            q = e / jnp.sum(e, axis=-1, keepdims=True)
        o[...] = q

    return pl.pallas_call(
        body,
        grid=(N // BN,),
        in_specs=[
            pl.BlockSpec((BN, D), lambda i: (i, 0)),
            pl.BlockSpec((BN, D), lambda i: (i, 0)),
            pl.BlockSpec((BN, D), lambda i: (i, 0)),
            pl.BlockSpec((BN, 1), lambda i: (i, 0)),
            pl.BlockSpec((BN, 1), lambda i: (i, 0)),
        ],
        out_specs=pl.BlockSpec((BN, D), lambda i: (i, 0)),
        out_shape=jax.ShapeDtypeStruct((N, D), jnp.float32),
    )


@functools.lru_cache(maxsize=None)
def _make_sc_pass(Npad, D, Epad, with_deg):
    EPT = Epad // NW
    CHUNKS = EPT // CH
    RZ = Npad // 16
    mesh = plsc.VectorSubcoreMesh(core_axis_name="c", subcore_axis_name="s")

    outs = [jax.ShapeDtypeStruct((2, Npad, D), jnp.float32)]
    scratch = [
        pltpu.VMEM((IR, RI), jnp.int32),
        pltpu.VMEM((CH, D), jnp.float32),
        pltpu.VMEM((CH, D), jnp.float32),
        pltpu.VMEM((CH, D), jnp.float32),
        pltpu.VMEM_SHARED((Npad, D), jnp.float32),
        pltpu.SemaphoreType.DMA,
        pltpu.SemaphoreType.DMA,
        pltpu.SemaphoreType.DMA,
        pltpu.SemaphoreType.DMA,
        pltpu.SemaphoreType.DMA,
        pltpu.SemaphoreType.DMA,
        pltpu.SemaphoreType.DMA,
        pltpu.SemaphoreType.DMA,
        pltpu.SemaphoreType.DMA,
    ]
    if with_deg:
        outs.append(jax.ShapeDtypeStruct((2 * Npad,), jnp.float32))
        scratch += [
            pltpu.VMEM((CH,), jnp.float32),
            pltpu.VMEM_SHARED((Npad,), jnp.float32),
            pltpu.VMEM((RZ,), jnp.float32),
            pltpu.SemaphoreType.DMA,
        ]

    def body(q_hbm, w_hbm, idx_hbm, *rest):
        if with_deg:
            (z_hbm, z1_hbm, agg_out, deg_out,
             idx_ring, w0, q0, q1, agg_sh,
             a0, b0, b1, c0, c1,
             i0, i1, i2, i3,
             ones_v, deg_sh, deg_v, semD) = rest
        else:
            (z_hbm, agg_out,
             idx_ring, w0, q0, q1, agg_sh,
             a0, b0, b1, c0, c1,
             i0, i1, i2, i3) = rest
        qb = (q0, q1)
        semB = (b0, b1)
        semC = (c0, c1)
        semI = (i0, i1, i2, i3)

        c = lax.axis_index("c")
        s = lax.axis_index("s")
        wid = c * 16 + s
        zb = pl.multiple_of(s * RZ, 8)

        pltpu.sync_copy(z_hbm.at[pl.ds(zb, RZ)], agg_sh.at[pl.ds(zb, RZ)])
        if with_deg:
            pltpu.sync_copy(z1_hbm.at[pl.ds(zb, RZ)], deg_v)
            pltpu.sync_copy(deg_v, deg_sh.at[pl.ds(zb, RZ)])
            for i in range(CH // 16):
                ones_v[pl.ds(i * 16, 16)] = jnp.full((16,), 1.0, jnp.float32)
        plsc.subcore_barrier()

        base0 = wid * EPT

        def issue_idx(g, k):
            pltpu.async_copy(idx_hbm.at[wid, pl.ds(g, 1)],
                             idx_ring.at[pl.ds(k, 1)], semI[k])

        def wait_idx(k):
            pltpu.make_async_copy(idx_hbm.at[0, pl.ds(0, 1)],
                                  idx_ring.at[pl.ds(k, 1)], semI[k]).wait()

        def issue_w(g):
            pltpu.async_copy(w_hbm.at[pl.ds(base0 + g * CH, CH)], w0, a0)

        def wait_w():
            pltpu.make_async_copy(w_hbm.at[pl.ds(0, CH)], w0, a0).wait()

        def issue_q(g, kq, ki):
            pltpu.async_copy(q_hbm.at[idx_ring.at[ki, pl.ds(0, CH)]],
                             qb[kq], semB[kq])

        def wait_q(kq):
            pltpu.make_async_copy(w_hbm.at[pl.ds(0, CH)], qb[kq], semB[kq]).wait()

        def wait_sc(kq):
            pltpu.make_async_copy(w_hbm.at[pl.ds(0, CH)], qb[kq], semC[kq]).wait()

        issue_idx(0, 0)
        issue_idx(1, 1)
        issue_idx(2, 2)
        issue_w(0)
        wait_idx(0)
        issue_q(0, 0, 0)

        def phase(g, p):
            kq = p % QR
            ko = (p + 1) % QR
            ki = p % IR
            wait_q(kq)
            wait_w()

            @plsc.parallel_loop(0, CH, step=1, unroll=8)
            def _mul(r):
                for cc in range(D // 16):
                    sl = pl.ds(cc * 16, 16)
                    qb[kq][r, sl] = qb[kq][r, sl] * w0[r, sl]

            @pl.when(g + 1 < CHUNKS)
            def _():
                issue_w(g + 1)
            @pl.when(g >= 1)
            def _():
                wait_sc(ko)
                if with_deg:
                    pltpu.make_async_copy(z1_hbm.at[pl.ds(0, CH)], ones_v,
                                          semD).wait()

            @pl.when(g + 3 < CHUNKS)
            def _():
                issue_idx(g + 3, (p + 3) % IR)

            @pl.when(g + 1 < CHUNKS)
            def _():
                wait_idx((p + 1) % IR)
                issue_q(g + 1, ko, (p + 1) % IR)
            pltpu.async_copy(qb[kq], agg_sh.at[idx_ring.at[ki, pl.ds(128, CH)]],
                             semC[kq], add=True)
            if with_deg:
                pltpu.async_copy(ones_v,
                                 deg_sh.at[idx_ring.at[ki, pl.ds(128, CH)]],
                                 semD, add=True)

        def macro(m, cr):
            g0 = m * IR
            for p in range(IR):
                phase(g0 + p, p)
            return cr
        lax.fori_loop(0, CHUNKS // IR, macro, 0)

        wait_sc((CHUNKS - 1) % QR)
        if with_deg:
            pltpu.make_async_copy(z1_hbm.at[pl.ds(0, CH)], ones_v, semD).wait()
        plsc.subcore_barrier()

        pltpu.sync_copy(agg_sh.at[pl.ds(zb, RZ)], agg_out.at[c, pl.ds(zb, RZ)])
        if with_deg:
            db = pl.multiple_of(c * Npad + zb, 8)
            pltpu.sync_copy(deg_sh.at[pl.ds(zb, RZ)], deg_v)
            pltpu.sync_copy(deg_v, deg_out.at[pl.ds(db, RZ)])

    return pl.kernel(body, mesh=mesh, out_type=outs, scratch_types=scratch)


def kernel(input, edge_index, edge_attr, W1, b1, W2, b2):
    N, D = input.shape
    E, DE = edge_attr.shape
    H = W1.shape[1]
    CB = NW * CH * IR
    Epad = ((E + CB - 1) // CB) * CB
    CHUNKS = Epad // (NW * CH)

    Npad = ((N + 127) // 128) * 128

    ea_p = jnp.pad(edge_attr, ((0, Epad - E), (0, 0)))
    src = jnp.pad(edge_index[0], (0, Epad - E)).reshape(NW, CHUNKS, CH)
    dst = jnp.pad(edge_index[1], (0, Epad - E),
                  constant_values=N).reshape(NW, CHUNKS, CH)
    src = jnp.pad(src, ((0, 0), (0, 0), (0, 128 - CH)))
    dst = jnp.pad(dst, ((0, 0), (0, 0), (0, 128 - CH)), constant_values=N)
    idx = jnp.concatenate([src, dst], axis=2)
    z = jnp.zeros((Npad, D), jnp.float32)
    z1 = jnp.zeros((Npad,), jnp.float32)

    w_pad = _make_fnet(E, Epad, DE, H, D)(
        ea_p, W1, b1.reshape(1, H), W2, b2.reshape(1, D))

    BN = 2000 if N % 2000 == 0 else N
    q0 = _make_softmax(N, D, BN)(input)

    agg1, deg = _make_sc_pass(Npad, D, Epad, True)(q0, w_pad, idx, z, z1)
    agg1 = agg1[:, :N]
    deg = deg.reshape(2, Npad)[:, :N].reshape(2, N, 1)
    q1 = _make_update(N, D, BN, True)(input, agg1[0], agg1[1], deg[0], deg[1])

    (agg2,) = _make_sc_pass(Npad, D, Epad, False)(q1, w_pad, idx, z)
    agg2 = agg2[:, :N]
    out = _make_update(N, D, BN, False)(input, agg2[0], agg2[1], deg[0], deg[1])
    return out


# speedup vs baseline: 1.4197x; 1.4197x over previous
import functools

import jax
import jax.numpy as jnp
from jax import lax
from jax.experimental import pallas as pl
from jax.experimental.pallas import tpu as pltpu
from jax.experimental.pallas import tpu_sc as plsc

CH = 128
NW = 32
RI = 512


@functools.lru_cache(maxsize=None)
def _make_fnet(E, Epad, DE, H, D):
    BE = 2048
    grid = (Epad // BE,)

    def body(ea, w1, b1, w2, b2, w_out):
        h = jnp.tanh(jnp.dot(ea[...], w1[...], preferred_element_type=jnp.float32)
                     + b1[...])
        w = jnp.dot(h, w2[...], preferred_element_type=jnp.float32) + b2[...]
        i = pl.program_id(0)
        rows = i * BE + lax.broadcasted_iota(jnp.int32, (BE, 1), 0)
        w_out[...] = jnp.where(rows < E, w, 0.0)

    return pl.pallas_call(
        body,
        grid=grid,
        in_specs=[
            pl.BlockSpec((BE, DE), lambda i: (i, 0)),
            pl.BlockSpec((DE, H), lambda i: (0, 0)),
            pl.BlockSpec((1, H), lambda i: (0, 0)),
            pl.BlockSpec((H, D), lambda i: (0, 0)),
            pl.BlockSpec((1, D), lambda i: (0, 0)),
        ],
        out_specs=pl.BlockSpec((BE, D), lambda i: (i, 0)),
        out_shape=jax.ShapeDtypeStruct((Epad, D), jnp.float32),
    )


@functools.lru_cache(maxsize=None)
def _make_softmax(N, D, BN):
    def body(x, o):
        v = x[...]
        m = jnp.max(v, axis=-1, keepdims=True)
        e = jnp.exp(v - m)
        o[...] = e / jnp.sum(e, axis=-1, keepdims=True)

    return pl.pallas_call(
        body,
        grid=(N // BN,),
        in_specs=[pl.BlockSpec((BN, D), lambda i: (i, 0))],
        out_specs=pl.BlockSpec((BN, D), lambda i: (i, 0)),
        out_shape=jax.ShapeDtypeStruct((N, D), jnp.float32),
    )


@functools.lru_cache(maxsize=None)
def _make_update(N, D, BN, do_softmax):
    def body(x, p0, p1, d0, d1, o):
        deg = d0[...] + d1[...]
        degc = jnp.maximum(deg, 1.0)
        q = x[...] - (p0[...] + p1[...]) / degc
        if do_softmax:
            m = jnp.max(q, axis=-1, keepdims=True)
            e = jnp.exp(q - m)
            q = e / jnp.sum(e, axis=-1, keepdims=True)
        o[...] = q

    return pl.pallas_call(
        body,
        grid=(N // BN,),
        in_specs=[
            pl.BlockSpec((BN, D), lambda i: (i, 0)),
            pl.BlockSpec((BN, D), lambda i: (i, 0)),
            pl.BlockSpec((BN, D), lambda i: (i, 0)),
            pl.BlockSpec((BN, 1), lambda i: (i, 0)),
            pl.BlockSpec((BN, 1), lambda i: (i, 0)),
        ],
        out_specs=pl.BlockSpec((BN, D), lambda i: (i, 0)),
        out_shape=jax.ShapeDtypeStruct((N, D), jnp.float32),
    )


@functools.lru_cache(maxsize=None)
def _make_sc_pass(Npad, D, Epad, with_deg):
    EPT = Epad // NW
    CHUNKS = EPT // CH
    RZ = Npad // 16
    mesh = plsc.VectorSubcoreMesh(core_axis_name="c", subcore_axis_name="s")

    outs = [jax.ShapeDtypeStruct((2, Npad, D), jnp.float32)]
    scratch = [
        pltpu.VMEM((1, RI), jnp.int32),
        pltpu.VMEM((CH, D), jnp.float32),
        pltpu.VMEM((CH, D), jnp.float32),
        pltpu.VMEM_SHARED((Npad, D), jnp.float32),
        pltpu.SemaphoreType.DMA,
        pltpu.SemaphoreType.DMA,
        pltpu.SemaphoreType.DMA,
    ]
    if with_deg:
        outs.append(jax.ShapeDtypeStruct((2 * Npad,), jnp.float32))
        scratch += [
            pltpu.VMEM((CH,), jnp.float32),
            pltpu.VMEM_SHARED((Npad,), jnp.float32),
            pltpu.VMEM((RZ,), jnp.float32),
            pltpu.SemaphoreType.DMA,
        ]

    def body(q_hbm, w_hbm, idx_hbm, *rest):
        if with_deg:
            (z_hbm, z1_hbm, agg_out, deg_out,
             idx_b, w0, q0, agg_sh,
             a0, b0, c0,
             ones_v, deg_sh, deg_v, semD) = rest
        else:
            (z_hbm, agg_out,
             idx_b, w0, q0, agg_sh,
             a0, b0, c0) = rest

        c = lax.axis_index("c")
        s = lax.axis_index("s")
        wid = c * 16 + s
        zb = pl.multiple_of(s * RZ, 8)

        pltpu.sync_copy(z_hbm.at[pl.ds(zb, RZ)], agg_sh.at[pl.ds(zb, RZ)])
        if with_deg:
            pltpu.sync_copy(z1_hbm.at[pl.ds(zb, RZ)], deg_v)
            pltpu.sync_copy(deg_v, deg_sh.at[pl.ds(zb, RZ)])
            for i in range(CH // 16):
                ones_v[pl.ds(i * 16, 16)] = jnp.full((16,), 1.0, jnp.float32)
        plsc.subcore_barrier()

        base0 = wid * EPT

        def chunk(g, cr):
            @pl.when(g >= 1)
            def _():
                pltpu.make_async_copy(w_hbm.at[pl.ds(0, CH)], q0, c0).wait()
                if with_deg:
                    pltpu.make_async_copy(z1_hbm.at[pl.ds(0, CH)], ones_v,
                                          semD).wait()
            pltpu.sync_copy(idx_hbm.at[wid, pl.ds(g, 1)], idx_b)
            pltpu.async_copy(w_hbm.at[pl.ds(base0 + g * CH, CH)], w0, a0)
            pltpu.async_copy(q_hbm.at[idx_b.at[0, pl.ds(0, CH)]], q0, b0)
            pltpu.make_async_copy(w_hbm.at[pl.ds(0, CH)], q0, b0).wait()
            pltpu.make_async_copy(w_hbm.at[pl.ds(0, CH)], w0, a0).wait()

            @plsc.parallel_loop(0, CH, step=1, unroll=8)
            def _mul(r):
                for cc in range(D // 16):
                    sl = pl.ds(cc * 16, 16)
                    q0[r, sl] = q0[r, sl] * w0[r, sl]

            pltpu.async_copy(q0, agg_sh.at[idx_b.at[0, pl.ds(256, CH)]],
                             c0, add=True)
            if with_deg:
                pltpu.async_copy(ones_v,
                                 deg_sh.at[idx_b.at[0, pl.ds(256, CH)]],
                                 semD, add=True)
            return cr
        lax.fori_loop(0, CHUNKS, chunk, 0)

        pltpu.make_async_copy(w_hbm.at[pl.ds(0, CH)], q0, c0).wait()
        if with_deg:
            pltpu.make_async_copy(z1_hbm.at[pl.ds(0, CH)], ones_v, semD).wait()
        plsc.subcore_barrier()

        pltpu.sync_copy(agg_sh.at[pl.ds(zb, RZ)], agg_out.at[c, pl.ds(zb, RZ)])
        if with_deg:
            db = pl.multiple_of(c * Npad + zb, 8)
            pltpu.sync_copy(deg_sh.at[pl.ds(zb, RZ)], deg_v)
            pltpu.sync_copy(deg_v, deg_out.at[pl.ds(db, RZ)])

    return pl.kernel(body, mesh=mesh, out_type=outs, scratch_types=scratch)


def kernel(input, edge_index, edge_attr, W1, b1, W2, b2):
    N, D = input.shape
    E, DE = edge_attr.shape
    H = W1.shape[1]
    CB = NW * CH
    Epad = ((E + CB - 1) // CB) * CB
    CHUNKS = Epad // (NW * CH)

    Npad = ((N + 127) // 128) * 128

    ea_p = jnp.pad(edge_attr, ((0, Epad - E), (0, 0)))
    src = jnp.pad(edge_index[0], (0, Epad - E)).reshape(NW, CHUNKS, CH)
    dst = jnp.pad(edge_index[1], (0, Epad - E),
                  constant_values=N).reshape(NW, CHUNKS, CH)
    src = jnp.pad(src, ((0, 0), (0, 0), (0, 256 - CH)))
    dst = jnp.pad(dst, ((0, 0), (0, 0), (0, 256 - CH)), constant_values=N)
    idx = jnp.concatenate([src, dst], axis=2)
    z = jnp.zeros((Npad, D), jnp.float32)
    z1 = jnp.zeros((Npad,), jnp.float32)

    w_pad = _make_fnet(E, Epad, DE, H, D)(
        ea_p, W1, b1.reshape(1, H), W2, b2.reshape(1, D))

    BN = 2000 if N % 2000 == 0 else N
    q0 = _make_softmax(N, D, BN)(input)

    agg1, deg = _make_sc_pass(Npad, D, Epad, True)(q0, w_pad, idx, z, z1)
    agg1 = agg1[:, :N]
    deg = deg.reshape(2, Npad)[:, :N].reshape(2, N, 1)
    q1 = _make_update(N, D, BN, True)(input, agg1[0], agg1[1], deg[0], deg[1])

    (agg2,) = _make_sc_pass(Npad, D, Epad, False)(q1, w_pad, idx, z)
    agg2 = agg2[:, :N]
    out = _make_update(N, D, BN, False)(input, agg2[0], agg2[1], deg[0], deg[1])
    return out
